# baseline (speedup 1.0000x reference)
import jax
import jax.numpy as jnp
from jax.experimental import pallas as pl
from jax.experimental.pallas import tpu as pltpu

_ANCHOR_LENGTHS = (1.0, 2.0, 3.0, 4.0, 5.0, 7.0, 9.0)
_A = len(_ANCHOR_LENGTHS)


def _anchors_1d(Lf):
    lengths = jnp.array(_ANCHOR_LENGTHS, dtype=jnp.float32)
    centers = jnp.arange(Lf, dtype=jnp.float32) + 0.5
    c = jnp.broadcast_to(centers[:, None], (Lf, _A))
    w = jnp.broadcast_to(lengths[None, :], (Lf, _A))
    return jnp.stack([c - 0.5 * w, c + 0.5 * w], axis=-1).reshape(Lf * _A, 2)


def _rpn_kernel(f_ref, wt_ref, cb_ref, wo_ref, bo_ref, wr_ref, br_ref,
                obj_ref, reg_ref):
    f = f_ref[0].astype(jnp.bfloat16)
    C, L = f.shape
    g0 = jax.lax.dot(wt_ref[1], f, preferred_element_type=jnp.float32)
    gm = jax.lax.dot(wt_ref[0], f, preferred_element_type=jnp.float32)
    gp = jax.lax.dot(wt_ref[2], f, preferred_element_type=jnp.float32)
    zero_col = jnp.zeros((C, 1), dtype=jnp.float32)
    h = g0
    h = h + jnp.concatenate([zero_col, gm[:, :-1]], axis=1)
    h = h + jnp.concatenate([gp[:, 1:], zero_col], axis=1)
    h = jnp.maximum(h + cb_ref[...], 0.0)
    dn = (((0,), (0,)), ((), ()))
    obj = jax.lax.dot_general(h, wo_ref[...], dn,
                              preferred_element_type=jnp.float32)
    reg = jax.lax.dot_general(h, wr_ref[...], dn,
                              preferred_element_type=jnp.float32)
    obj_ref[0] = obj + bo_ref[...]
    reg_ref[0] = reg + br_ref[...]


def kernel(feat, conv_w, conv_b, w_obj, b_obj, w_reg, b_reg):
    B, C, Lf = feat.shape
    A, R = w_obj.shape[0], w_reg.shape[0]
    w_taps = jnp.transpose(conv_w, (2, 0, 1)).astype(jnp.bfloat16)
    cb = conv_b[:, None]
    wo = w_obj.T
    wr = w_reg.T
    bo = b_obj[None, :]
    br = b_reg[None, :]
    obj, reg = pl.pallas_call(
        _rpn_kernel,
        grid=(B,),
        in_specs=[
            pl.BlockSpec((1, C, Lf), lambda b: (b, 0, 0)),
            pl.BlockSpec((3, C, C), lambda b: (0, 0, 0)),
            pl.BlockSpec((C, 1), lambda b: (0, 0)),
            pl.BlockSpec((C, A), lambda b: (0, 0)),
            pl.BlockSpec((1, A), lambda b: (0, 0)),
            pl.BlockSpec((C, R), lambda b: (0, 0)),
            pl.BlockSpec((1, R), lambda b: (0, 0)),
        ],
        out_specs=[
            pl.BlockSpec((1, Lf, A), lambda b: (b, 0, 0)),
            pl.BlockSpec((1, Lf, R), lambda b: (b, 0, 0)),
        ],
        out_shape=[
            jax.ShapeDtypeStruct((B, Lf, A), jnp.float32),
            jax.ShapeDtypeStruct((B, Lf, R), jnp.float32),
        ],
        compiler_params=pltpu.CompilerParams(
            dimension_semantics=("parallel",)),
    )(feat, w_taps, cb, wo, bo, wr, br)
    return (obj.reshape(B, Lf * A), reg.reshape(B, Lf * A, 2),
            _anchors_1d(Lf))


# speedup vs baseline: 1.0443x; 1.0443x over previous
import jax
import jax.numpy as jnp
from jax.experimental import pallas as pl
from jax.experimental.pallas import tpu as pltpu

_ANCHOR_LENGTHS = (1.0, 2.0, 3.0, 4.0, 5.0, 7.0, 9.0)
_A = len(_ANCHOR_LENGTHS)


def _anchors_1d(Lf):
    lengths = jnp.array(_ANCHOR_LENGTHS, dtype=jnp.float32)
    centers = jnp.arange(Lf, dtype=jnp.float32) + 0.5
    c = jnp.broadcast_to(centers[:, None], (Lf, _A))
    w = jnp.broadcast_to(lengths[None, :], (Lf, _A))
    return jnp.stack([c - 0.5 * w, c + 0.5 * w], axis=-1).reshape(Lf * _A, 2)


def _rpn_kernel(f_ref, wt_ref, cb_ref, wo_ref, bo_ref, wr_ref, br_ref,
                obj_ref, reg_ref):
    ft = jnp.transpose(f_ref[0].astype(jnp.bfloat16))
    L, C = ft.shape
    g0 = jax.lax.dot(ft, wt_ref[1], preferred_element_type=jnp.float32)
    gm = jax.lax.dot(ft, wt_ref[0], preferred_element_type=jnp.float32)
    gp = jax.lax.dot(ft, wt_ref[2], preferred_element_type=jnp.float32)
    zero_row = jnp.zeros((1, C), dtype=jnp.float32)
    h = g0
    h = h + jnp.concatenate([zero_row, gm[:-1, :]], axis=0)
    h = h + jnp.concatenate([gp[1:, :], zero_row], axis=0)
    h = jnp.maximum(h + cb_ref[...], 0.0)
    obj = jax.lax.dot(h, wo_ref[...], preferred_element_type=jnp.float32)
    reg = jax.lax.dot(h, wr_ref[...], preferred_element_type=jnp.float32)
    obj_ref[0] = obj + bo_ref[...]
    reg_ref[0] = reg + br_ref[...]


def kernel(feat, conv_w, conv_b, w_obj, b_obj, w_reg, b_reg):
    B, C, Lf = feat.shape
    A, R = w_obj.shape[0], w_reg.shape[0]
    w_taps = jnp.transpose(conv_w, (2, 1, 0)).astype(jnp.bfloat16)
    cb = conv_b[None, :]
    wo = w_obj.T
    wr = w_reg.T
    bo = b_obj[None, :]
    br = b_reg[None, :]
    obj, reg = pl.pallas_call(
        _rpn_kernel,
        grid=(B,),
        in_specs=[
            pl.BlockSpec((1, C, Lf), lambda b: (b, 0, 0)),
            pl.BlockSpec((3, C, C), lambda b: (0, 0, 0)),
            pl.BlockSpec((1, C), lambda b: (0, 0)),
            pl.BlockSpec((C, A), lambda b: (0, 0)),
            pl.BlockSpec((1, A), lambda b: (0, 0)),
            pl.BlockSpec((C, R), lambda b: (0, 0)),
            pl.BlockSpec((1, R), lambda b: (0, 0)),
        ],
        out_specs=[
            pl.BlockSpec((1, Lf, A), lambda b: (b, 0, 0)),
            pl.BlockSpec((1, Lf, R), lambda b: (b, 0, 0)),
        ],
        out_shape=[
            jax.ShapeDtypeStruct((B, Lf, A), jnp.float32),
            jax.ShapeDtypeStruct((B, Lf, R), jnp.float32),
        ],
        compiler_params=pltpu.CompilerParams(
            dimension_semantics=("parallel",)),
    )(feat, w_taps, cb, wo, bo, wr, br)
    return (obj.reshape(B, Lf * A), reg.reshape(B, Lf * A, 2),
            _anchors_1d(Lf))
